# baseline (speedup 1.0000x reference)
import functools

import jax
import jax.numpy as jnp
from jax import lax
from jax.experimental import pallas as pl
from jax.experimental.pallas import tpu as pltpu
from jax.experimental.pallas import tpu_sc as plsc

_N_ROWS = 16384
_D = 4096
_NC = 2
_NS = 16
_L = 16
_NW = _NC * _NS
_ROWS_PER_W = _N_ROWS // _NW
_RBLK = 4
_NBLK = _ROWS_PER_W // _RBLK
_NCHUNK = _D // _L


def _make_sc_perm():
    mesh = plsc.VectorSubcoreMesh(core_axis_name="c", subcore_axis_name="s")

    @functools.partial(
        pl.kernel,
        mesh=mesh,
        compiler_params=pltpu.CompilerParams(needs_layout_passes=False),
        out_type=jax.ShapeDtypeStruct((_N_ROWS, _D), jnp.float32),
        scratch_types=(
            [pltpu.VMEM((_D,), jnp.int32)]
            + [pltpu.VMEM((_D,), jnp.float32) for _ in range(_RBLK)]
            + [pltpu.VMEM((_D,), jnp.float32) for _ in range(_RBLK)]
        ),
    )
    def k(x_hbm, perm_hbm, out_hbm, perm_v, *rows):
        in_v = rows[:_RBLK]
        out_v = rows[_RBLK:]
        wid = lax.axis_index("s") * _NC + lax.axis_index("c")
        base = wid * _ROWS_PER_W
        pltpu.sync_copy(perm_hbm, perm_v)

        def blk_body(b, carry):
            row0 = base + b * _RBLK
            for r in range(_RBLK):
                pltpu.sync_copy(x_hbm.at[row0 + r], in_v[r])

            def chunk_body(j, c):
                sl = pl.ds(j * _L, _L)
                idx = perm_v[sl]
                for r in range(_RBLK):
                    out_v[r][sl] = plsc.load_gather(in_v[r], [idx])
                return c

            lax.fori_loop(0, _NCHUNK, chunk_body, 0)
            for r in range(_RBLK):
                pltpu.sync_copy(out_v[r], out_hbm.at[row0 + r])
            return carry

        lax.fori_loop(0, _NBLK, blk_body, 0)

    return k


_sc_perm = _make_sc_perm()


def kernel(x, perm):
    out = _sc_perm(x, perm)
    return (out, 0)
